# baseline (speedup 1.0000x reference)
import functools

import jax
import jax.numpy as jnp
from jax import lax
from jax.experimental import pallas as pl
from jax.experimental.pallas import tpu as pltpu
from jax.experimental.pallas import tpu_sc as plsc

_L = 16

_mesh = plsc.VectorSubcoreMesh(
    core_axis_name="c", subcore_axis_name="s", num_cores=1
)


@functools.partial(
    pl.kernel,
    mesh=_mesh,
    out_type=jax.ShapeDtypeStruct((64,), jnp.float32),
    scratch_types=[
        pltpu.VMEM((64,), jnp.float32),
        pltpu.VMEM((64,), jnp.float32),
        pltpu.SemaphoreType.DMA,
    ],
)
def _sub_sc(a_hbm, b_hbm, out_hbm, a_v, b_v, sem):
    sid = lax.axis_index("s")

    @pl.when(sid == 0)
    def _():
        cp_a = pltpu.async_copy(a_hbm, a_v, sem)
        cp_b = pltpu.async_copy(b_hbm, b_v, sem)
        cp_a.wait()
        cp_b.wait()
        for i in range(64 // _L):
            sl = pl.ds(i * _L, _L)
            a_v[sl] = a_v[sl] - b_v[sl]
        pltpu.sync_copy(a_v, out_hbm)


@jax.jit
def kernel(a, b, c):
    del c
    return _sub_sc(a, b)


# speedup vs baseline: 1.0045x; 1.0045x over previous
import functools

import jax
import jax.numpy as jnp
from jax import lax
from jax.experimental import pallas as pl
from jax.experimental.pallas import tpu as pltpu
from jax.experimental.pallas import tpu_sc as plsc

_L = 16

_mesh = plsc.VectorSubcoreMesh(
    core_axis_name="c", subcore_axis_name="s", num_cores=1, num_subcores=1
)


@functools.partial(
    pl.kernel,
    mesh=_mesh,
    out_type=jax.ShapeDtypeStruct((64,), jnp.float32),
    scratch_types=[
        pltpu.VMEM((64,), jnp.float32),
        pltpu.VMEM((64,), jnp.float32),
        pltpu.SemaphoreType.DMA,
    ],
)
def _sub_sc(a_hbm, b_hbm, out_hbm, a_v, b_v, sem):
    cp_a = pltpu.async_copy(a_hbm, a_v, sem)
    cp_b = pltpu.async_copy(b_hbm, b_v, sem)
    cp_a.wait()
    cp_b.wait()
    for i in range(64 // _L):
        sl = pl.ds(i * _L, _L)
        a_v[sl] = a_v[sl] - b_v[sl]
    pltpu.sync_copy(a_v, out_hbm)


@jax.jit
def kernel(a, b, c):
    del c
    return _sub_sc(a, b)


# speedup vs baseline: 15.6225x; 15.5519x over previous
import jax
import jax.numpy as jnp
from jax.experimental import pallas as pl


def _sub_tc(a_ref, b_ref, o_ref):
    o_ref[...] = a_ref[...] - b_ref[...]


@jax.jit
def kernel(a, b, c):
    del c
    return pl.pallas_call(
        _sub_tc,
        out_shape=jax.ShapeDtypeStruct((64,), jnp.float32),
    )(a, b)
